# baseline (speedup 1.0000x reference)
import functools

import jax
import jax.numpy as jnp
from jax.experimental import pallas as pl
from jax.experimental.pallas import tpu as pltpu

_B = 2
_S = 2048
_D = 768
_H = 1024
_RH = 128
_E = 8

_TS = 256
_RTS = 512


def _nt_dot(a, b):
    return jax.lax.dot_general(
        a, b, (((1,), (1,)), ((), ())), preferred_element_type=jnp.float32
    )


def _router_kernel(x_ref, rW1_ref, rb1_ref, rW2_ref, rb2_ref,
                   idx_ref, w_ref, acc_ref):
    i = pl.program_id(0)

    @pl.when(i == 0)
    def _init():
        acc_ref[...] = jnp.zeros_like(acc_ref)

    acc_ref[...] += jnp.sum(x_ref[...], axis=1)

    @pl.when(i == pl.num_programs(0) - 1)
    def _finish():
        pooled = acc_ref[...] * (1.0 / _S)
        rh = jnp.maximum(_nt_dot(pooled, rW1_ref[...]) + rb1_ref[...], 0.0)
        logits = _nt_dot(rh, rW2_ref[...]) + rb2_ref[...]
        m = jnp.max(logits, axis=1, keepdims=True)
        eg = jnp.exp(logits - m)
        gate = eg / jnp.sum(eg, axis=1, keepdims=True)
        eiota = jax.lax.broadcasted_iota(jnp.int32, gate.shape, 1)
        v1 = jnp.max(gate, axis=1, keepdims=True)
        i1 = jnp.min(jnp.where(gate >= v1, eiota, _E), axis=1, keepdims=True)
        masked = jnp.where(eiota == i1, -jnp.float32(jnp.inf), gate)
        v2 = jnp.max(masked, axis=1, keepdims=True)
        i2 = jnp.min(jnp.where(masked >= v2, eiota, _E), axis=1, keepdims=True)
        idx_ref[...] = jnp.concatenate([i1, i2], axis=1).astype(jnp.int32)
        w_ref[...] = jnp.concatenate([v1, v2], axis=1)


def _router(x, rW1, rb1, rW2, rb2):
    n_blocks = _S // _RTS
    idx, w = pl.pallas_call(
        _router_kernel,
        grid=(n_blocks,),
        in_specs=[
            pl.BlockSpec((_B, _RTS, _D), lambda i: (0, i, 0)),
            pl.BlockSpec((_RH, _D), lambda i: (0, 0)),
            pl.BlockSpec((1, _RH), lambda i: (0, 0)),
            pl.BlockSpec((_E, _RH), lambda i: (0, 0)),
            pl.BlockSpec((1, _E), lambda i: (0, 0)),
        ],
        out_specs=[
            pl.BlockSpec((_B, 2), lambda i: (0, 0)),
            pl.BlockSpec((_B, 2), lambda i: (0, 0)),
        ],
        out_shape=[
            jax.ShapeDtypeStruct((_B, 2), jnp.int32),
            jax.ShapeDtypeStruct((_B, 2), jnp.float32),
        ],
        scratch_shapes=[pltpu.VMEM((_B, _D), jnp.float32)],
    )(x, rW1, rb1.reshape(1, _RH), rW2, rb2.reshape(1, _E))
    return idx, w


def _gather_cast_kernel(idxf_ref, w1_ref, w2_ref, w3_ref,
                        o1_ref, o2_ref, o3_ref):
    o1_ref[...] = w1_ref[...].astype(jnp.bfloat16)
    o2_ref[...] = w2_ref[...].astype(jnp.bfloat16)
    o3_ref[...] = w3_ref[...].astype(jnp.bfloat16)


def _gather_cast(eW1, eW2, eW3, idx):
    idxf = idx.reshape(2 * _B)
    grid_spec = pltpu.PrefetchScalarGridSpec(
        num_scalar_prefetch=1,
        grid=(2 * _B,),
        in_specs=[
            pl.BlockSpec((1, _H, _D), lambda p, idxf: (idxf[p], 0, 0)),
            pl.BlockSpec((1, _H, _H), lambda p, idxf: (idxf[p], 0, 0)),
            pl.BlockSpec((1, _D, _H), lambda p, idxf: (idxf[p], 0, 0)),
        ],
        out_specs=[
            pl.BlockSpec((1, _H, _D), lambda p, idxf: (p, 0, 0)),
            pl.BlockSpec((1, _H, _H), lambda p, idxf: (p, 0, 0)),
            pl.BlockSpec((1, _D, _H), lambda p, idxf: (p, 0, 0)),
        ],
    )
    return pl.pallas_call(
        _gather_cast_kernel,
        grid_spec=grid_spec,
        out_shape=[
            jax.ShapeDtypeStruct((2 * _B, _H, _D), jnp.bfloat16),
            jax.ShapeDtypeStruct((2 * _B, _H, _H), jnp.bfloat16),
            jax.ShapeDtypeStruct((2 * _B, _D, _H), jnp.bfloat16),
        ],
    )(idxf, eW1, eW2, eW3)


def _moe_kernel(idx_ref, x_ref,
                w1a_ref, w1b_ref, w2a_ref, w2b_ref, w3a_ref, w3b_ref,
                b1a_ref, b1b_ref, b2a_ref, b2b_ref, b3a_ref, b3b_ref,
                gw_ref, o_ref):
    b = pl.program_id(0)
    xb = x_ref[0].astype(jnp.bfloat16)

    def expert_p(w1_ref, w2_ref, w3_ref, b1_ref, b2_ref, b3_ref):
        h1 = jnp.maximum(_nt_dot(xb, w1_ref[0]) + b1_ref[0], 0.0)
        h2 = jnp.maximum(_nt_dot(h1.astype(jnp.bfloat16), w2_ref[0])
                         + b2_ref[0], 0.0)
        out = _nt_dot(h2.astype(jnp.bfloat16), w3_ref[0]) + b3_ref[0]
        m = jnp.max(out, axis=1, keepdims=True)
        eo = jnp.exp(out - m)
        return eo / jnp.sum(eo, axis=1, keepdims=True)

    pa = expert_p(w1a_ref, w2a_ref, w3a_ref, b1a_ref, b2a_ref, b3a_ref)
    pb = expert_p(w1b_ref, w2b_ref, w3b_ref, b1b_ref, b2b_ref, b3b_ref)
    o_ref[0] = gw_ref[b, 0] * pa + gw_ref[b, 1] * pb


def _moe_top2(x, eW1, eb1, eW2, eb2, eW3, eb3, idx, gw):
    w1c, w2c, w3c = _gather_cast(eW1, eW2, eW3, idx)
    n_s = _S // _TS
    grid_spec = pltpu.PrefetchScalarGridSpec(
        num_scalar_prefetch=1,
        grid=(_B, n_s),
        in_specs=[
            pl.BlockSpec((1, _TS, _D), lambda b, s, idx: (b, s, 0)),
            pl.BlockSpec((1, _H, _D), lambda b, s, idx: (2 * b, 0, 0)),
            pl.BlockSpec((1, _H, _D), lambda b, s, idx: (2 * b + 1, 0, 0)),
            pl.BlockSpec((1, _H, _H), lambda b, s, idx: (2 * b, 0, 0)),
            pl.BlockSpec((1, _H, _H), lambda b, s, idx: (2 * b + 1, 0, 0)),
            pl.BlockSpec((1, _D, _H), lambda b, s, idx: (2 * b, 0, 0)),
            pl.BlockSpec((1, _D, _H), lambda b, s, idx: (2 * b + 1, 0, 0)),
            pl.BlockSpec((1, 1, _H), lambda b, s, idx: (idx[b, 0], 0, 0)),
            pl.BlockSpec((1, 1, _H), lambda b, s, idx: (idx[b, 1], 0, 0)),
            pl.BlockSpec((1, 1, _H), lambda b, s, idx: (idx[b, 0], 0, 0)),
            pl.BlockSpec((1, 1, _H), lambda b, s, idx: (idx[b, 1], 0, 0)),
            pl.BlockSpec((1, 1, _D), lambda b, s, idx: (idx[b, 0], 0, 0)),
            pl.BlockSpec((1, 1, _D), lambda b, s, idx: (idx[b, 1], 0, 0)),
            pl.BlockSpec(memory_space=pltpu.SMEM),
        ],
        out_specs=pl.BlockSpec((1, _TS, _D), lambda b, s, idx: (b, s, 0)),
    )
    return pl.pallas_call(
        _moe_kernel,
        grid_spec=grid_spec,
        out_shape=jax.ShapeDtypeStruct((_B, _S, _D), jnp.float32),
    )(idx, x, w1c, w1c, w2c, w2c, w3c, w3c,
      eb1.reshape(_E, 1, _H), eb1.reshape(_E, 1, _H),
      eb2.reshape(_E, 1, _H), eb2.reshape(_E, 1, _H),
      eb3.reshape(_E, 1, _D), eb3.reshape(_E, 1, _D), gw)


def kernel(x, rW1, rb1, rW2, rb2, eW1, eb1, eW2, eb2, eW3, eb3):
    idx, gw = _router(x, rW1, rb1, rW2, rb2)
    return _moe_top2(x, eW1, eb1, eW2, eb2, eW3, eb3, idx, gw)


# speedup vs baseline: 1.1654x; 1.1654x over previous
import functools

import jax
import jax.numpy as jnp
from jax.experimental import pallas as pl
from jax.experimental.pallas import tpu as pltpu

_B = 2
_S = 2048
_D = 768
_H = 1024
_RH = 128
_E = 8

_TS = 256
_RTS = 512


def _nt_dot(a, b):
    return jax.lax.dot_general(
        a, b, (((1,), (1,)), ((), ())), preferred_element_type=jnp.float32
    )


def _router_kernel(x_ref, rW1_ref, rb1_ref, rW2_ref, rb2_ref,
                   idx_ref, w_ref, acc_ref):
    i = pl.program_id(0)

    @pl.when(i == 0)
    def _init():
        acc_ref[...] = jnp.zeros_like(acc_ref)

    acc_ref[...] += jnp.sum(x_ref[...], axis=1)

    @pl.when(i == pl.num_programs(0) - 1)
    def _finish():
        pooled = acc_ref[...] * (1.0 / _S)
        rh = jnp.maximum(_nt_dot(pooled, rW1_ref[...]) + rb1_ref[...], 0.0)
        logits = _nt_dot(rh, rW2_ref[...]) + rb2_ref[...]
        m = jnp.max(logits, axis=1, keepdims=True)
        eg = jnp.exp(logits - m)
        gate = eg / jnp.sum(eg, axis=1, keepdims=True)
        eiota = jax.lax.broadcasted_iota(jnp.int32, gate.shape, 1)
        v1 = jnp.max(gate, axis=1, keepdims=True)
        i1 = jnp.min(jnp.where(gate >= v1, eiota, _E), axis=1, keepdims=True)
        masked = jnp.where(eiota == i1, -jnp.float32(jnp.inf), gate)
        v2 = jnp.max(masked, axis=1, keepdims=True)
        i2 = jnp.min(jnp.where(masked >= v2, eiota, _E), axis=1, keepdims=True)
        idx_ref[...] = jnp.concatenate([i1, i2], axis=1).astype(jnp.int32)
        w_ref[...] = jnp.concatenate([v1, v2], axis=1)


def _router(x, rW1, rb1, rW2, rb2):
    n_blocks = _S // _RTS
    idx, w = pl.pallas_call(
        _router_kernel,
        grid=(n_blocks,),
        in_specs=[
            pl.BlockSpec((_B, _RTS, _D), lambda i: (0, i, 0)),
            pl.BlockSpec((_RH, _D), lambda i: (0, 0)),
            pl.BlockSpec((1, _RH), lambda i: (0, 0)),
            pl.BlockSpec((_E, _RH), lambda i: (0, 0)),
            pl.BlockSpec((1, _E), lambda i: (0, 0)),
        ],
        out_specs=[
            pl.BlockSpec((_B, 2), lambda i: (0, 0)),
            pl.BlockSpec((_B, 2), lambda i: (0, 0)),
        ],
        out_shape=[
            jax.ShapeDtypeStruct((_B, 2), jnp.int32),
            jax.ShapeDtypeStruct((_B, 2), jnp.float32),
        ],
        scratch_shapes=[pltpu.VMEM((_B, _D), jnp.float32)],
    )(x, rW1, rb1.reshape(1, _RH), rW2, rb2.reshape(1, _E))
    return idx, w


def _moe_kernel(idx_ref, x_ref,
                w1a_ref, w1b_ref, w2a_ref, w2b_ref, w3a_ref, w3b_ref,
                b1a_ref, b1b_ref, b2a_ref, b2b_ref, b3a_ref, b3b_ref,
                gw_ref, o_ref):
    b = pl.program_id(0)
    xb = x_ref[0]

    def expert_p(w1_ref, w2_ref, w3_ref, b1_ref, b2_ref, b3_ref):
        h1 = jnp.maximum(_nt_dot(xb, w1_ref[0]) + b1_ref[0], 0.0)
        h2 = jnp.maximum(_nt_dot(h1, w2_ref[0]) + b2_ref[0], 0.0)
        out = _nt_dot(h2, w3_ref[0]) + b3_ref[0]
        m = jnp.max(out, axis=1, keepdims=True)
        eo = jnp.exp(out - m)
        return eo / jnp.sum(eo, axis=1, keepdims=True)

    pa = expert_p(w1a_ref, w2a_ref, w3a_ref, b1a_ref, b2a_ref, b3a_ref)
    pb = expert_p(w1b_ref, w2b_ref, w3b_ref, b1b_ref, b2b_ref, b3b_ref)
    o_ref[0] = gw_ref[b, 0] * pa + gw_ref[b, 1] * pb


def _moe_top2(x, eW1, eb1, eW2, eb2, eW3, eb3, idx, gw):
    n_s = _S // _TS
    grid_spec = pltpu.PrefetchScalarGridSpec(
        num_scalar_prefetch=1,
        grid=(_B, n_s),
        in_specs=[
            pl.BlockSpec((1, _TS, _D), lambda b, s, idx: (b, s, 0)),
            pl.BlockSpec((1, _H, _D), lambda b, s, idx: (idx[b, 0], 0, 0)),
            pl.BlockSpec((1, _H, _D), lambda b, s, idx: (idx[b, 1], 0, 0)),
            pl.BlockSpec((1, _H, _H), lambda b, s, idx: (idx[b, 0], 0, 0)),
            pl.BlockSpec((1, _H, _H), lambda b, s, idx: (idx[b, 1], 0, 0)),
            pl.BlockSpec((1, _D, _H), lambda b, s, idx: (idx[b, 0], 0, 0)),
            pl.BlockSpec((1, _D, _H), lambda b, s, idx: (idx[b, 1], 0, 0)),
            pl.BlockSpec((1, 1, _H), lambda b, s, idx: (idx[b, 0], 0, 0)),
            pl.BlockSpec((1, 1, _H), lambda b, s, idx: (idx[b, 1], 0, 0)),
            pl.BlockSpec((1, 1, _H), lambda b, s, idx: (idx[b, 0], 0, 0)),
            pl.BlockSpec((1, 1, _H), lambda b, s, idx: (idx[b, 1], 0, 0)),
            pl.BlockSpec((1, 1, _D), lambda b, s, idx: (idx[b, 0], 0, 0)),
            pl.BlockSpec((1, 1, _D), lambda b, s, idx: (idx[b, 1], 0, 0)),
            pl.BlockSpec(memory_space=pltpu.SMEM),
        ],
        out_specs=pl.BlockSpec((1, _TS, _D), lambda b, s, idx: (b, s, 0)),
    )
    return pl.pallas_call(
        _moe_kernel,
        grid_spec=grid_spec,
        out_shape=jax.ShapeDtypeStruct((_B, _S, _D), jnp.float32),
    )(idx, x, eW1, eW1, eW2, eW2, eW3, eW3,
      eb1.reshape(_E, 1, _H), eb1.reshape(_E, 1, _H),
      eb2.reshape(_E, 1, _H), eb2.reshape(_E, 1, _H),
      eb3.reshape(_E, 1, _D), eb3.reshape(_E, 1, _D), gw)


def kernel(x, rW1, rb1, rW2, rb2, eW1, eb1, eW2, eb2, eW3, eb3):
    idx, gw = _router(x, rW1, rb1, rW2, rb2)
    return _moe_top2(x, eW1, eb1, eW2, eb2, eW3, eb3, idx, gw)


# speedup vs baseline: 1.2829x; 1.1008x over previous
import functools

import jax
import jax.numpy as jnp
from jax.experimental import pallas as pl
from jax.experimental.pallas import tpu as pltpu

_B = 2
_S = 2048
_D = 768
_H = 1024
_RH = 128
_E = 8

_TS = 512
_RTS = 512


def _nt_dot(a, b):
    return jax.lax.dot_general(
        a, b, (((1,), (1,)), ((), ())), preferred_element_type=jnp.float32
    )


def _router_kernel(x_ref, rW1_ref, rb1_ref, rW2_ref, rb2_ref,
                   idx_ref, w_ref, acc_ref):
    i = pl.program_id(0)

    @pl.when(i == 0)
    def _init():
        acc_ref[...] = jnp.zeros_like(acc_ref)

    acc_ref[...] += jnp.sum(x_ref[...], axis=1)

    @pl.when(i == pl.num_programs(0) - 1)
    def _finish():
        pooled = acc_ref[...] * (1.0 / _S)
        rh = jnp.maximum(_nt_dot(pooled, rW1_ref[...]) + rb1_ref[...], 0.0)
        logits = _nt_dot(rh, rW2_ref[...]) + rb2_ref[...]
        m = jnp.max(logits, axis=1, keepdims=True)
        eg = jnp.exp(logits - m)
        gate = eg / jnp.sum(eg, axis=1, keepdims=True)
        eiota = jax.lax.broadcasted_iota(jnp.int32, gate.shape, 1)
        v1 = jnp.max(gate, axis=1, keepdims=True)
        i1 = jnp.min(jnp.where(gate >= v1, eiota, _E), axis=1, keepdims=True)
        masked = jnp.where(eiota == i1, -jnp.float32(jnp.inf), gate)
        v2 = jnp.max(masked, axis=1, keepdims=True)
        i2 = jnp.min(jnp.where(masked >= v2, eiota, _E), axis=1, keepdims=True)
        idx_ref[...] = jnp.concatenate([i1, i2], axis=1).astype(jnp.int32)
        w_ref[...] = jnp.concatenate([v1, v2], axis=1)


def _router(x, rW1, rb1, rW2, rb2):
    n_blocks = _S // _RTS
    idx, w = pl.pallas_call(
        _router_kernel,
        grid=(n_blocks,),
        in_specs=[
            pl.BlockSpec((_B, _RTS, _D), lambda i: (0, i, 0)),
            pl.BlockSpec((_RH, _D), lambda i: (0, 0)),
            pl.BlockSpec((1, _RH), lambda i: (0, 0)),
            pl.BlockSpec((_E, _RH), lambda i: (0, 0)),
            pl.BlockSpec((1, _E), lambda i: (0, 0)),
        ],
        out_specs=[
            pl.BlockSpec((_B, 2), lambda i: (0, 0)),
            pl.BlockSpec((_B, 2), lambda i: (0, 0)),
        ],
        out_shape=[
            jax.ShapeDtypeStruct((_B, 2), jnp.int32),
            jax.ShapeDtypeStruct((_B, 2), jnp.float32),
        ],
        scratch_shapes=[pltpu.VMEM((_B, _D), jnp.float32)],
    )(x, rW1, rb1.reshape(1, _RH), rW2, rb2.reshape(1, _E))
    return idx, w


def _moe_kernel(idx_ref, x_ref,
                w1a_ref, w1b_ref, w2a_ref, w2b_ref, w3a_ref, w3b_ref,
                b1a_ref, b1b_ref, b2a_ref, b2b_ref, b3a_ref, b3b_ref,
                gw_ref, o_ref):
    b = pl.program_id(0)
    xb = x_ref[0]

    def expert_p(w1_ref, w2_ref, w3_ref, b1_ref, b2_ref, b3_ref):
        h1 = jnp.maximum(_nt_dot(xb, w1_ref[0]) + b1_ref[0], 0.0)
        h2 = jnp.maximum(_nt_dot(h1, w2_ref[0]) + b2_ref[0], 0.0)
        out = _nt_dot(h2, w3_ref[0]) + b3_ref[0]
        m = jnp.max(out, axis=1, keepdims=True)
        eo = jnp.exp(out - m)
        return eo / jnp.sum(eo, axis=1, keepdims=True)

    pa = expert_p(w1a_ref, w2a_ref, w3a_ref, b1a_ref, b2a_ref, b3a_ref)
    pb = expert_p(w1b_ref, w2b_ref, w3b_ref, b1b_ref, b2b_ref, b3b_ref)
    o_ref[0] = gw_ref[b, 0] * pa + gw_ref[b, 1] * pb


def _moe_top2(x, eW1, eb1, eW2, eb2, eW3, eb3, idx, gw):
    n_s = _S // _TS
    grid_spec = pltpu.PrefetchScalarGridSpec(
        num_scalar_prefetch=1,
        grid=(_B, n_s),
        in_specs=[
            pl.BlockSpec((1, _TS, _D), lambda b, s, idx: (b, s, 0)),
            pl.BlockSpec((1, _H, _D), lambda b, s, idx: (idx[b, 0], 0, 0)),
            pl.BlockSpec((1, _H, _D), lambda b, s, idx: (idx[b, 1], 0, 0)),
            pl.BlockSpec((1, _H, _H), lambda b, s, idx: (idx[b, 0], 0, 0)),
            pl.BlockSpec((1, _H, _H), lambda b, s, idx: (idx[b, 1], 0, 0)),
            pl.BlockSpec((1, _D, _H), lambda b, s, idx: (idx[b, 0], 0, 0)),
            pl.BlockSpec((1, _D, _H), lambda b, s, idx: (idx[b, 1], 0, 0)),
            pl.BlockSpec((1, 1, _H), lambda b, s, idx: (idx[b, 0], 0, 0)),
            pl.BlockSpec((1, 1, _H), lambda b, s, idx: (idx[b, 1], 0, 0)),
            pl.BlockSpec((1, 1, _H), lambda b, s, idx: (idx[b, 0], 0, 0)),
            pl.BlockSpec((1, 1, _H), lambda b, s, idx: (idx[b, 1], 0, 0)),
            pl.BlockSpec((1, 1, _D), lambda b, s, idx: (idx[b, 0], 0, 0)),
            pl.BlockSpec((1, 1, _D), lambda b, s, idx: (idx[b, 1], 0, 0)),
            pl.BlockSpec(memory_space=pltpu.SMEM),
        ],
        out_specs=pl.BlockSpec((1, _TS, _D), lambda b, s, idx: (b, s, 0)),
    )
    return pl.pallas_call(
        _moe_kernel,
        grid_spec=grid_spec,
        out_shape=jax.ShapeDtypeStruct((_B, _S, _D), jnp.float32),
    )(idx, x, eW1, eW1, eW2, eW2, eW3, eW3,
      eb1.reshape(_E, 1, _H), eb1.reshape(_E, 1, _H),
      eb2.reshape(_E, 1, _H), eb2.reshape(_E, 1, _H),
      eb3.reshape(_E, 1, _D), eb3.reshape(_E, 1, _D), gw)


def kernel(x, rW1, rb1, rW2, rb2, eW1, eb1, eW2, eb2, eW3, eb3):
    idx, gw = _router(x, rW1, rb1, rW2, rb2)
    return _moe_top2(x, eW1, eb1, eW2, eb2, eW3, eb3, idx, gw)


# speedup vs baseline: 1.3343x; 1.0401x over previous
import functools

import jax
import jax.numpy as jnp
from jax.experimental import pallas as pl
from jax.experimental.pallas import tpu as pltpu

_B = 2
_S = 2048
_D = 768
_H = 1024
_RH = 128
_E = 8

_TS = 1024
_RTS = 512


def _nt_dot(a, b):
    return jax.lax.dot_general(
        a, b, (((1,), (1,)), ((), ())), preferred_element_type=jnp.float32
    )


def _router_kernel(x_ref, rW1_ref, rb1_ref, rW2_ref, rb2_ref,
                   idx_ref, w_ref, acc_ref):
    i = pl.program_id(0)

    @pl.when(i == 0)
    def _init():
        acc_ref[...] = jnp.zeros_like(acc_ref)

    acc_ref[...] += jnp.sum(x_ref[...], axis=1)

    @pl.when(i == pl.num_programs(0) - 1)
    def _finish():
        pooled = acc_ref[...] * (1.0 / _S)
        rh = jnp.maximum(_nt_dot(pooled, rW1_ref[...]) + rb1_ref[...], 0.0)
        logits = _nt_dot(rh, rW2_ref[...]) + rb2_ref[...]
        m = jnp.max(logits, axis=1, keepdims=True)
        eg = jnp.exp(logits - m)
        gate = eg / jnp.sum(eg, axis=1, keepdims=True)
        eiota = jax.lax.broadcasted_iota(jnp.int32, gate.shape, 1)
        v1 = jnp.max(gate, axis=1, keepdims=True)
        i1 = jnp.min(jnp.where(gate >= v1, eiota, _E), axis=1, keepdims=True)
        masked = jnp.where(eiota == i1, -jnp.float32(jnp.inf), gate)
        v2 = jnp.max(masked, axis=1, keepdims=True)
        i2 = jnp.min(jnp.where(masked >= v2, eiota, _E), axis=1, keepdims=True)
        idx_ref[...] = jnp.concatenate([i1, i2], axis=1).astype(jnp.int32)
        w_ref[...] = jnp.concatenate([v1, v2], axis=1)


def _router(x, rW1, rb1, rW2, rb2):
    n_blocks = _S // _RTS
    idx, w = pl.pallas_call(
        _router_kernel,
        grid=(n_blocks,),
        in_specs=[
            pl.BlockSpec((_B, _RTS, _D), lambda i: (0, i, 0)),
            pl.BlockSpec((_RH, _D), lambda i: (0, 0)),
            pl.BlockSpec((1, _RH), lambda i: (0, 0)),
            pl.BlockSpec((_E, _RH), lambda i: (0, 0)),
            pl.BlockSpec((1, _E), lambda i: (0, 0)),
        ],
        out_specs=[
            pl.BlockSpec((_B, 2), lambda i: (0, 0)),
            pl.BlockSpec((_B, 2), lambda i: (0, 0)),
        ],
        out_shape=[
            jax.ShapeDtypeStruct((_B, 2), jnp.int32),
            jax.ShapeDtypeStruct((_B, 2), jnp.float32),
        ],
        scratch_shapes=[pltpu.VMEM((_B, _D), jnp.float32)],
    )(x, rW1, rb1.reshape(1, _RH), rW2, rb2.reshape(1, _E))
    return idx, w


def _moe_kernel(idx_ref, x_ref,
                w1a_ref, w1b_ref, w2a_ref, w2b_ref, w3a_ref, w3b_ref,
                b1a_ref, b1b_ref, b2a_ref, b2b_ref, b3a_ref, b3b_ref,
                gw_ref, o_ref):
    b = pl.program_id(0)
    xb = x_ref[0]

    def expert_p(w1_ref, w2_ref, w3_ref, b1_ref, b2_ref, b3_ref):
        h1 = jnp.maximum(_nt_dot(xb, w1_ref[0]) + b1_ref[0], 0.0)
        h2 = jnp.maximum(_nt_dot(h1, w2_ref[0]) + b2_ref[0], 0.0)
        out = _nt_dot(h2, w3_ref[0]) + b3_ref[0]
        m = jnp.max(out, axis=1, keepdims=True)
        eo = jnp.exp(out - m)
        return eo / jnp.sum(eo, axis=1, keepdims=True)

    o_ref[0] = gw_ref[b, 0] * expert_p(w1a_ref, w2a_ref, w3a_ref,
                                       b1a_ref, b2a_ref, b3a_ref)
    o_ref[0] += gw_ref[b, 1] * expert_p(w1b_ref, w2b_ref, w3b_ref,
                                        b1b_ref, b2b_ref, b3b_ref)


def _moe_top2(x, eW1, eb1, eW2, eb2, eW3, eb3, idx, gw):
    n_s = _S // _TS
    grid_spec = pltpu.PrefetchScalarGridSpec(
        num_scalar_prefetch=1,
        grid=(_B, n_s),
        in_specs=[
            pl.BlockSpec((1, _TS, _D), lambda b, s, idx: (b, s, 0)),
            pl.BlockSpec((1, _H, _D), lambda b, s, idx: (idx[b, 0], 0, 0)),
            pl.BlockSpec((1, _H, _D), lambda b, s, idx: (idx[b, 1], 0, 0)),
            pl.BlockSpec((1, _H, _H), lambda b, s, idx: (idx[b, 0], 0, 0)),
            pl.BlockSpec((1, _H, _H), lambda b, s, idx: (idx[b, 1], 0, 0)),
            pl.BlockSpec((1, _D, _H), lambda b, s, idx: (idx[b, 0], 0, 0)),
            pl.BlockSpec((1, _D, _H), lambda b, s, idx: (idx[b, 1], 0, 0)),
            pl.BlockSpec((1, 1, _H), lambda b, s, idx: (idx[b, 0], 0, 0)),
            pl.BlockSpec((1, 1, _H), lambda b, s, idx: (idx[b, 1], 0, 0)),
            pl.BlockSpec((1, 1, _H), lambda b, s, idx: (idx[b, 0], 0, 0)),
            pl.BlockSpec((1, 1, _H), lambda b, s, idx: (idx[b, 1], 0, 0)),
            pl.BlockSpec((1, 1, _D), lambda b, s, idx: (idx[b, 0], 0, 0)),
            pl.BlockSpec((1, 1, _D), lambda b, s, idx: (idx[b, 1], 0, 0)),
            pl.BlockSpec(memory_space=pltpu.SMEM),
        ],
        out_specs=pl.BlockSpec((1, _TS, _D), lambda b, s, idx: (b, s, 0)),
    )
    return pl.pallas_call(
        _moe_kernel,
        grid_spec=grid_spec,
        out_shape=jax.ShapeDtypeStruct((_B, _S, _D), jnp.float32),
    )(idx, x, eW1, eW1, eW2, eW2, eW3, eW3,
      eb1.reshape(_E, 1, _H), eb1.reshape(_E, 1, _H),
      eb2.reshape(_E, 1, _H), eb2.reshape(_E, 1, _H),
      eb3.reshape(_E, 1, _D), eb3.reshape(_E, 1, _D), gw)


def kernel(x, rW1, rb1, rW2, rb2, eW1, eb1, eW2, eb2, eW3, eb3):
    idx, gw = _router(x, rW1, rb1, rW2, rb2)
    return _moe_top2(x, eW1, eb1, eW2, eb2, eW3, eb3, idx, gw)
